# Initial kernel scaffold; baseline (speedup 1.0000x reference)
import jax
import jax.numpy as jnp
from jax.experimental import pallas as pl


def kernel(inputs, targets, features_bank):
    raise NotImplementedError("write your pallas kernel here")


# speedup vs baseline: 1.5187x; 1.5187x over previous
import jax
import jax.numpy as jnp
from jax import lax
from jax.experimental import pallas as pl
from jax.experimental.pallas import tpu as pltpu

_NUM_CLASSES = 100000
_NUM_FEATURES = 64
_BATCH = 1024
_BLK = 2000
_NBLK = _NUM_CLASSES // _BLK
_INV_TEMP = 20.0


def _loss_kernel(x_ref, t_ref, bank_ref, out_ref, ni_ref, m_ref, s_ref, tl_ref):
    j = pl.program_id(0)

    @pl.when(j == 0)
    def _init():
        x = x_ref[...]
        nrm = jnp.sqrt(jnp.sum(x * x, axis=1, keepdims=True))
        ni_ref[...] = (x * (_INV_TEMP / jnp.maximum(nrm, 1e-12))).astype(jnp.bfloat16)
        m_ref[...] = jnp.full((1, _BATCH), -1e30, jnp.float32)
        s_ref[...] = jnp.zeros((1, _BATCH), jnp.float32)
        tl_ref[...] = jnp.zeros((1, _BATCH), jnp.float32)

    bank = bank_ref[...].astype(jnp.bfloat16)
    ni = ni_ref[...]
    logits = lax.dot_general(
        bank, ni, (((1,), (1,)), ((), ())),
        preferred_element_type=jnp.float32)
    m_old = m_ref[...]
    m_new = jnp.maximum(m_old, jnp.max(logits, axis=0, keepdims=True))
    p = jnp.exp(logits - m_new)
    s_ref[...] = s_ref[...] * jnp.exp(m_old - m_new) + jnp.sum(p, axis=0, keepdims=True)
    m_ref[...] = m_new
    col = j * _BLK + lax.broadcasted_iota(jnp.int32, (_BLK, _BATCH), 0)
    hit = col == t_ref[...]
    tl_ref[...] += jnp.sum(jnp.where(hit, logits, 0.0), axis=0, keepdims=True)

    @pl.when(j == _NBLK - 1)
    def _fin():
        lse = m_ref[...] + jnp.log(s_ref[...])
        out_ref[0, 0] = jnp.sum(lse - tl_ref[...]) * (1.0 / _BATCH)


def kernel(inputs, targets, features_bank):
    loss = pl.pallas_call(
        _loss_kernel,
        grid=(_NBLK,),
        in_specs=[
            pl.BlockSpec((_BATCH, _NUM_FEATURES), lambda j: (0, 0)),
            pl.BlockSpec((1, _BATCH), lambda j: (0, 0)),
            pl.BlockSpec((_BLK, _NUM_FEATURES), lambda j: (j, 0)),
        ],
        out_specs=pl.BlockSpec(memory_space=pltpu.SMEM),
        out_shape=jax.ShapeDtypeStruct((1, 1), jnp.float32),
        scratch_shapes=[
            pltpu.VMEM((_BATCH, _NUM_FEATURES), jnp.bfloat16),
            pltpu.VMEM((1, _BATCH), jnp.float32),
            pltpu.VMEM((1, _BATCH), jnp.float32),
            pltpu.VMEM((1, _BATCH), jnp.float32),
        ],
    )(inputs, targets.astype(jnp.int32).reshape(1, _BATCH), features_bank)
    return loss[0, 0]
